# baseline (speedup 1.0000x reference)
import functools

import jax
import jax.numpy as jnp
from jax import lax
from jax.experimental import pallas as pl
from jax.experimental.pallas import tpu as pltpu
from jax.experimental.pallas import tpu_sc as plsc

B, S, H, E, F = 2, 2048, 768, 8, 3072
T = B * S
TB = 256
NB = T // TB + E
NP = NB * TB
NBP = 32
CH = 512

NC, NS = 2, 16
NPC = NP // NC
RPT = NPC // NS
TPT = T // (NC * NS)
GCH = 64


SCH = 512
NSC = NP // SCH


def _router_body(x_ref, wg_ref, gs_ref, pos_ref, be_ref, bv_ref,
                 tok_ref, gsort_ref, loss_ref):
    x = x_ref[...]
    wg = wg_ref[...]
    logits = lax.dot_general(x, wg, (((1,), (1,)), ((), ())),
                             preferred_element_type=jnp.float32)
    m = jnp.max(logits, axis=-1, keepdims=True)
    ex = jnp.exp(logits - m)
    gs = ex / jnp.sum(ex, axis=-1, keepdims=True)
    gs_ref[...] = gs
    top = jnp.max(gs, axis=-1, keepdims=True)
    lanes = lax.broadcasted_iota(jnp.int32, (T, E), 1)
    eid = jnp.min(jnp.where(gs == top, lanes, E), axis=-1, keepdims=True)
    g = top / (top + 1e-08)
    onehot = (lanes == eid).astype(jnp.float32)
    usage = jnp.sum(onehot, axis=0, keepdims=True) / T
    probs = jnp.sum(gs, axis=0, keepdims=True) / T
    loss_ref[...] = (E * jnp.sum(probs * usage)).reshape(1, 1)

    tri = (lax.broadcasted_iota(jnp.int32, (CH, CH), 0)
           >= lax.broadcasted_iota(jnp.int32, (CH, CH), 1)).astype(jnp.float32)
    run = jnp.zeros((1, E), jnp.float32)
    rank_chunks = []
    for ci in range(T // CH):
        oh = lax.slice(onehot, (ci * CH, 0), ((ci + 1) * CH, E))
        rank = lax.dot_general(tri, oh, (((1,), (0,)), ((), ())),
                               preferred_element_type=jnp.float32) + run
        run = lax.slice(rank, (CH - 1, 0), (CH, E))
        rank_chunks.append(rank)
    ranks = jnp.concatenate(rank_chunks, axis=0)
    counts = run
    padded = jnp.ceil(counts / TB) * TB
    lt8 = (lax.broadcasted_iota(jnp.int32, (E, E), 0)
           < lax.broadcasted_iota(jnp.int32, (E, E), 1)).astype(jnp.float32)
    base = lax.dot_general(padded, lt8, (((1,), (0,)), ((), ())),
                           preferred_element_type=jnp.float32)
    rank_sel = jnp.sum(onehot * ranks, axis=-1, keepdims=True)
    base_sel = jnp.sum(onehot * base, axis=-1, keepdims=True)
    posf = base_sel + rank_sel - 1.0
    pos_ref[...] = posf.astype(jnp.int32)

    tf = lax.broadcasted_iota(jnp.int32, (T, 1), 0).astype(jnp.float32)
    for cb in range(NSC):
        slot_i = lax.broadcasted_iota(jnp.int32, (1, SCH), 1) + cb * SCH
        slots = slot_i.astype(jnp.float32)
        mask = (posf == slots).astype(jnp.float32)
        hit = jnp.sum(mask, axis=0, keepdims=True)
        pad_tok = ((slot_i % T).astype(jnp.float32)) * (1.0 - hit)
        tok_ref[cb:cb + 1, :] = (jnp.sum(mask * tf, axis=0, keepdims=True)
                                 + pad_tok).astype(jnp.int32)
        gsort_ref[cb:cb + 1, :] = jnp.sum(mask * g, axis=0, keepdims=True)

    ends = base + padded
    bs = lax.broadcasted_iota(jnp.int32, (NBP, 1), 0).astype(jnp.float32) * TB
    ge = (bs >= ends).astype(jnp.float32)
    raw = jnp.sum(ge, axis=-1, keepdims=True)
    total = jnp.sum(padded)
    be_ref[...] = jnp.minimum(raw, E - 1.0).astype(jnp.int32)
    bv_ref[...] = (bs < total).astype(jnp.int32)


NF = 2
FC = F // NF


def _ffn_body(be_ref, bv_ref, x_ref, w1_ref, b1_ref, w2_ref, b2_ref, g_ref,
              out_ref):
    b = pl.program_id(0)
    f = pl.program_id(1)

    @pl.when(bv_ref[b] != 0)
    def _compute():
        h = lax.dot_general(x_ref[...], w1_ref[0], (((1,), (1,)), ((), ())),
                            preferred_element_type=jnp.float32)
        h = jnp.maximum(h + b1_ref[0], 0.0)
        y = lax.dot_general(h, w2_ref[0], (((1,), (1,)), ((), ())),
                            preferred_element_type=jnp.float32)
        part = g_ref[...] * y

        @pl.when(f == 0)
        def _init():
            out_ref[...] = part + g_ref[...] * b2_ref[0]

        @pl.when(f != 0)
        def _acc():
            out_ref[...] += part


def _sc_gather_body(per_tile, idx_hbm, src_hbm, out_hbm, idx64, rows, sem):
    c = lax.axis_index("c")
    s = lax.axis_index("s")
    tbase = (c * NS + s) * per_tile

    def gloop(j, carry):
        o = tbase + j * GCH
        pltpu.sync_copy(idx_hbm.at[pl.ds(o, GCH)], idx64)
        pltpu.async_copy(src_hbm.at[idx64], rows, sem).wait()
        pltpu.sync_copy(rows, out_hbm.at[pl.ds(o, GCH)])
        return carry
    lax.fori_loop(0, per_tile // GCH, gloop, 0)


@functools.lru_cache(maxsize=1)
def _sc_kernels():
    mesh = plsc.VectorSubcoreMesh(core_axis_name="c", subcore_axis_name="s")

    def make_gather(nrows):
        per_tile = nrows // (NC * NS)
        return pl.kernel(
            functools.partial(_sc_gather_body, per_tile),
            out_type=jax.ShapeDtypeStruct((nrows, H), jnp.float32),
            mesh=mesh,
            scratch_types=(
                pltpu.VMEM((GCH,), jnp.int32),
                pltpu.VMEM((GCH, H), jnp.float32),
                pltpu.SemaphoreType.DMA,
            ),
            compiler_params=pltpu.CompilerParams(needs_layout_passes=False),
        )

    return make_gather(NP), make_gather(T)


@jax.jit
def kernel(x, Wg, W1, b1, W2, b2):
    x_flat = x.reshape(T, H)

    gs, pos2, be2, bv2, tok2, gsort2, loss = pl.pallas_call(
        _router_body,
        out_shape=(
            jax.ShapeDtypeStruct((T, E), jnp.float32),
            jax.ShapeDtypeStruct((T, 1), jnp.int32),
            jax.ShapeDtypeStruct((NBP, 1), jnp.int32),
            jax.ShapeDtypeStruct((NBP, 1), jnp.int32),
            jax.ShapeDtypeStruct((NSC, SCH), jnp.int32),
            jax.ShapeDtypeStruct((NSC, SCH), jnp.float32),
            jax.ShapeDtypeStruct((1, 1), jnp.float32),
        ),
    )(x_flat, Wg)
    pos = pos2.reshape(T)
    tok = tok2.reshape(NP)
    g_sorted = gsort2.reshape(NP)

    _sc_dispatch, _sc_combine = _sc_kernels()
    x_sorted = _sc_dispatch(tok, x_flat)

    y_sorted = pl.pallas_call(
        _ffn_body,
        grid_spec=pltpu.PrefetchScalarGridSpec(
            num_scalar_prefetch=2,
            grid=(NB, NF),
            in_specs=[
                pl.BlockSpec((TB, H), lambda b, f, be, bv: (b, 0)),
                pl.BlockSpec((1, FC, H), lambda b, f, be, bv: (be[b], f, 0)),
                pl.BlockSpec((1, 1, FC), lambda b, f, be, bv: (be[b], 0, f)),
                pl.BlockSpec((1, H, FC), lambda b, f, be, bv: (be[b], 0, f)),
                pl.BlockSpec((1, 1, H), lambda b, f, be, bv: (be[b], 0, 0)),
                pl.BlockSpec((TB, 1), lambda b, f, be, bv: (b, 0)),
            ],
            out_specs=pl.BlockSpec((TB, H), lambda b, f, be, bv: (b, 0)),
        ),
        out_shape=jax.ShapeDtypeStruct((NP, H), jnp.float32),
        compiler_params=pltpu.CompilerParams(
            dimension_semantics=("arbitrary", "arbitrary"),
        ),
    )(be2.reshape(NBP), bv2.reshape(NBP), x_sorted, W1,
      b1.reshape(E, 1, F), W2, b2.reshape(E, 1, H),
      g_sorted.reshape(NP, 1))

    out_flat = _sc_combine(pos, y_sorted)

    return out_flat.reshape(B, S, H), gs.reshape(B, S, E), loss.reshape(())


# speedup vs baseline: 1.3009x; 1.3009x over previous
import functools

import jax
import jax.numpy as jnp
from jax import lax
from jax.experimental import pallas as pl
from jax.experimental.pallas import tpu as pltpu
from jax.experimental.pallas import tpu_sc as plsc

B, S, H, E, F = 2, 2048, 768, 8, 3072
T = B * S
TB = 256
NB = T // TB + E
NP = NB * TB
NBP = 32
CH = 512

NC, NS = 2, 16
NPC = NP // NC
RPT = NPC // NS
TPT = T // (NC * NS)
GCH = 64


SCH = 512
NSC = NP // SCH


def _router_body(x_ref, wg_ref, gs_ref, pos_ref, be_ref, bv_ref,
                 tok_ref, g_ref, loss_ref):
    x = x_ref[...]
    wg = wg_ref[...]
    logits = lax.dot_general(x, wg, (((1,), (1,)), ((), ())),
                             preferred_element_type=jnp.float32)
    m = jnp.max(logits, axis=-1, keepdims=True)
    ex = jnp.exp(logits - m)
    gs = ex / jnp.sum(ex, axis=-1, keepdims=True)
    gs_ref[...] = gs
    top = jnp.max(gs, axis=-1, keepdims=True)
    lanes = lax.broadcasted_iota(jnp.int32, (T, E), 1)
    eid = jnp.min(jnp.where(gs == top, lanes, E), axis=-1, keepdims=True)
    g_ref[...] = top / (top + 1e-08)
    onehot = (lanes == eid).astype(jnp.float32)
    usage = jnp.sum(onehot, axis=0, keepdims=True) / T
    probs = jnp.sum(gs, axis=0, keepdims=True) / T
    loss_ref[...] = (E * jnp.sum(probs * usage)).reshape(1, 1)

    tri = (lax.broadcasted_iota(jnp.int32, (CH, CH), 0)
           >= lax.broadcasted_iota(jnp.int32, (CH, CH), 1)).astype(jnp.float32)
    run = jnp.zeros((1, E), jnp.float32)
    rank_chunks = []
    for ci in range(T // CH):
        oh = lax.slice(onehot, (ci * CH, 0), ((ci + 1) * CH, E))
        rank = lax.dot_general(tri, oh, (((1,), (0,)), ((), ())),
                               preferred_element_type=jnp.float32) + run
        run = lax.slice(rank, (CH - 1, 0), (CH, E))
        rank_chunks.append(rank)
    ranks = jnp.concatenate(rank_chunks, axis=0)
    counts = run
    padded = jnp.ceil(counts / TB) * TB
    lt8 = (lax.broadcasted_iota(jnp.int32, (E, E), 0)
           < lax.broadcasted_iota(jnp.int32, (E, E), 1)).astype(jnp.float32)
    base = lax.dot_general(padded, lt8, (((1,), (0,)), ((), ())),
                           preferred_element_type=jnp.float32)
    rank_sel = jnp.sum(onehot * ranks, axis=-1, keepdims=True)
    base_sel = jnp.sum(onehot * base, axis=-1, keepdims=True)
    posf = base_sel + rank_sel - 1.0
    pos_ref[...] = posf.astype(jnp.int32)

    tf = lax.broadcasted_iota(jnp.int32, (T, 1), 0).astype(jnp.float32)
    for cb in range(NSC):
        slot_i = lax.broadcasted_iota(jnp.int32, (1, SCH), 1) + cb * SCH
        slots = slot_i.astype(jnp.float32)
        mask = (posf == slots).astype(jnp.float32)
        hit = jnp.sum(mask, axis=0, keepdims=True)
        pad_tok = ((slot_i % T).astype(jnp.float32)) * (1.0 - hit)
        tok_ref[cb:cb + 1, :] = (jnp.sum(mask * tf, axis=0, keepdims=True)
                                 + pad_tok).astype(jnp.int32)

    ends = base + padded
    bs = lax.broadcasted_iota(jnp.int32, (NBP, 1), 0).astype(jnp.float32) * TB
    ge = (bs >= ends).astype(jnp.float32)
    raw = jnp.sum(ge, axis=-1, keepdims=True)
    total = jnp.sum(padded)
    be_ref[...] = jnp.minimum(raw, E - 1.0).astype(jnp.int32)
    bv_ref[...] = (bs < total).astype(jnp.int32)


def _ffn_body(be_ref, bv_ref, x_ref, w1_ref, b1_ref, w2_ref, b2_ref, g_ref,
              out_ref):
    b = pl.program_id(0)

    @pl.when(bv_ref[b] != 0)
    def _compute():
        h = lax.dot_general(x_ref[...], w1_ref[0], (((1,), (1,)), ((), ())),
                            preferred_element_type=jnp.float32)
        h = jnp.maximum(h + b1_ref[0], 0.0)
        y = lax.dot_general(h, w2_ref[0], (((1,), (1,)), ((), ())),
                            preferred_element_type=jnp.float32)
        out_ref[...] = g_ref[...] * (y + b2_ref[0])


def _sc_gather_body(per_tile, idx_hbm, src_hbm, out_hbm, idx64, rows, sem):
    c = lax.axis_index("c")
    s = lax.axis_index("s")
    tbase = (c * NS + s) * per_tile

    def gloop(j, carry):
        o = tbase + j * GCH
        pltpu.sync_copy(idx_hbm.at[pl.ds(o, GCH)], idx64)
        pltpu.async_copy(src_hbm.at[idx64], rows, sem).wait()
        pltpu.sync_copy(rows, out_hbm.at[pl.ds(o, GCH)])
        return carry
    lax.fori_loop(0, per_tile // GCH, gloop, 0)


def _sc_dispatch_body(tok_hbm, x_hbm, g_hbm, xs_hbm, gsort_hbm,
                      idx64, rows, g_v, gso64, sem):
    c = lax.axis_index("c")
    s = lax.axis_index("s")
    per_tile = NP // (NC * NS)
    tbase = (c * NS + s) * per_tile
    pltpu.sync_copy(g_hbm, g_v)

    def gloop(j, carry):
        o = tbase + j * GCH
        pltpu.sync_copy(tok_hbm.at[pl.ds(o, GCH)], idx64)
        cp = pltpu.async_copy(x_hbm.at[idx64], rows, sem)
        for k in range(GCH // 16):
            tk = idx64[pl.ds(k * 16, 16)]
            gso64[pl.ds(k * 16, 16)] = plsc.load_gather(g_v, [tk])
        pltpu.sync_copy(gso64, gsort_hbm.at[pl.ds(o, GCH)])
        cp.wait()
        pltpu.sync_copy(rows, xs_hbm.at[pl.ds(o, GCH)])
        return carry
    lax.fori_loop(0, per_tile // GCH, gloop, 0)


@functools.lru_cache(maxsize=1)
def _sc_kernels():
    mesh = plsc.VectorSubcoreMesh(core_axis_name="c", subcore_axis_name="s")

    dispatch = pl.kernel(
        _sc_dispatch_body,
        out_type=(
            jax.ShapeDtypeStruct((NP, H), jnp.float32),
            jax.ShapeDtypeStruct((NP,), jnp.float32),
        ),
        mesh=mesh,
        scratch_types=(
            pltpu.VMEM((GCH,), jnp.int32),
            pltpu.VMEM((GCH, H), jnp.float32),
            pltpu.VMEM((T,), jnp.float32),
            pltpu.VMEM((GCH,), jnp.float32),
            pltpu.SemaphoreType.DMA,
        ),
        compiler_params=pltpu.CompilerParams(needs_layout_passes=False),
    )
    combine = pl.kernel(
        functools.partial(_sc_gather_body, T // (NC * NS)),
        out_type=jax.ShapeDtypeStruct((T, H), jnp.float32),
        mesh=mesh,
        scratch_types=(
            pltpu.VMEM((GCH,), jnp.int32),
            pltpu.VMEM((GCH, H), jnp.float32),
            pltpu.SemaphoreType.DMA,
        ),
        compiler_params=pltpu.CompilerParams(needs_layout_passes=False),
    )
    return dispatch, combine


@jax.jit
def kernel(x, Wg, W1, b1, W2, b2):
    x_flat = x.reshape(T, H)

    gs, pos2, be2, bv2, tok2, g2, loss = pl.pallas_call(
        _router_body,
        out_shape=(
            jax.ShapeDtypeStruct((T, E), jnp.float32),
            jax.ShapeDtypeStruct((T, 1), jnp.int32),
            jax.ShapeDtypeStruct((NBP, 1), jnp.int32),
            jax.ShapeDtypeStruct((NBP, 1), jnp.int32),
            jax.ShapeDtypeStruct((NSC, SCH), jnp.int32),
            jax.ShapeDtypeStruct((T, 1), jnp.float32),
            jax.ShapeDtypeStruct((1, 1), jnp.float32),
        ),
    )(x_flat, Wg)
    pos = pos2.reshape(T)
    tok = tok2.reshape(NP)

    _sc_dispatch, _sc_combine = _sc_kernels()
    x_sorted, g_sorted = _sc_dispatch(tok, x_flat, g2.reshape(T))

    y_sorted = pl.pallas_call(
        _ffn_body,
        grid_spec=pltpu.PrefetchScalarGridSpec(
            num_scalar_prefetch=2,
            grid=(NB,),
            in_specs=[
                pl.BlockSpec((TB, H), lambda b, be, bv: (b, 0)),
                pl.BlockSpec((1, F, H), lambda b, be, bv: (be[b], 0, 0)),
                pl.BlockSpec((1, 1, F), lambda b, be, bv: (be[b], 0, 0)),
                pl.BlockSpec((1, H, F), lambda b, be, bv: (be[b], 0, 0)),
                pl.BlockSpec((1, 1, H), lambda b, be, bv: (be[b], 0, 0)),
                pl.BlockSpec((TB, 1), lambda b, be, bv: (b, 0)),
            ],
            out_specs=pl.BlockSpec((TB, H), lambda b, be, bv: (b, 0)),
        ),
        out_shape=jax.ShapeDtypeStruct((NP, H), jnp.float32),
        compiler_params=pltpu.CompilerParams(
            dimension_semantics=("arbitrary",),
        ),
    )(be2.reshape(NBP), bv2.reshape(NBP), x_sorted, W1,
      b1.reshape(E, 1, F), W2, b2.reshape(E, 1, H),
      g_sorted.reshape(NP, 1))

    out_flat = _sc_combine(pos, y_sorted)

    return out_flat.reshape(B, S, H), gs.reshape(B, S, E), loss.reshape(())


# speedup vs baseline: 1.4928x; 1.1475x over previous
import functools

import jax
import jax.numpy as jnp
from jax import lax
from jax.experimental import pallas as pl
from jax.experimental.pallas import tpu as pltpu
from jax.experimental.pallas import tpu_sc as plsc

B, S, H, E, F = 2, 2048, 768, 8, 3072
T = B * S
TB = 256
NB = T // TB + E
NP = NB * TB
NBP = 32
CH = 512

NC, NS = 2, 16
NPC = NP // NC
RPT = NPC // NS
TPT = T // (NC * NS)
GCH = 64


def _router_body(x_ref, wg_ref, gs_ref, pos_ref, be_ref, bv_ref,
                 g_ref, loss_ref):
    x = x_ref[...]
    wg = wg_ref[...]
    logits = lax.dot_general(x, wg, (((1,), (1,)), ((), ())),
                             preferred_element_type=jnp.float32)
    m = jnp.max(logits, axis=-1, keepdims=True)
    ex = jnp.exp(logits - m)
    gs = ex / jnp.sum(ex, axis=-1, keepdims=True)
    gs_ref[...] = gs
    top = jnp.max(gs, axis=-1, keepdims=True)
    lanes = lax.broadcasted_iota(jnp.int32, (T, E), 1)
    eid = jnp.min(jnp.where(gs == top, lanes, E), axis=-1, keepdims=True)
    g_ref[...] = jnp.reshape(top / (top + 1e-08), (T // 128, 128))
    onehot = (lanes == eid).astype(jnp.float32)
    usage = jnp.sum(onehot, axis=0, keepdims=True) / T
    probs = jnp.sum(gs, axis=0, keepdims=True) / T
    loss_ref[...] = (E * jnp.sum(probs * usage)).reshape(1, 1)

    tri = (lax.broadcasted_iota(jnp.int32, (CH, CH), 0)
           >= lax.broadcasted_iota(jnp.int32, (CH, CH), 1)).astype(jnp.float32)
    run = jnp.zeros((1, E), jnp.float32)
    rank_chunks = []
    for ci in range(T // CH):
        oh = lax.slice(onehot, (ci * CH, 0), ((ci + 1) * CH, E))
        rank = lax.dot_general(tri, oh, (((1,), (0,)), ((), ())),
                               preferred_element_type=jnp.float32) + run
        run = lax.slice(rank, (CH - 1, 0), (CH, E))
        rank_chunks.append(rank)
    ranks = jnp.concatenate(rank_chunks, axis=0)
    counts = run
    padded = jnp.ceil(counts / TB) * TB
    lt8 = (lax.broadcasted_iota(jnp.int32, (E, E), 0)
           < lax.broadcasted_iota(jnp.int32, (E, E), 1)).astype(jnp.float32)
    base = lax.dot_general(padded, lt8, (((1,), (0,)), ((), ())),
                           preferred_element_type=jnp.float32)
    rank_sel = jnp.sum(onehot * ranks, axis=-1, keepdims=True)
    base_sel = jnp.sum(onehot * base, axis=-1, keepdims=True)
    posf = base_sel + rank_sel - 1.0
    pos_ref[...] = jnp.reshape(posf.astype(jnp.int32), (T // 128, 128))


    ends = base + padded
    bs = lax.broadcasted_iota(jnp.int32, (NBP, 1), 0).astype(jnp.float32) * TB
    ge = (bs >= ends).astype(jnp.float32)
    raw = jnp.sum(ge, axis=-1, keepdims=True)
    total = jnp.sum(padded)
    be_ref[...] = jnp.minimum(raw, E - 1.0).astype(jnp.int32)
    bv_ref[...] = (bs < total).astype(jnp.int32)


def _ffn_body(be_ref, bv_ref, x_ref, w1_ref, b1_ref, w2_ref, b2_ref, g_ref,
              out_ref):
    b = pl.program_id(0)

    @pl.when(bv_ref[b] != 0)
    def _compute():
        h = lax.dot_general(x_ref[...], w1_ref[0], (((1,), (1,)), ((), ())),
                            preferred_element_type=jnp.float32)
        h = jnp.maximum(h + b1_ref[0], 0.0)
        y = lax.dot_general(h, w2_ref[0], (((1,), (1,)), ((), ())),
                            preferred_element_type=jnp.float32)
        out_ref[...] = g_ref[...] * (y + b2_ref[0])


def _sc_combine_body(idx_hbm, src_hbm, out_hbm, idx_a, idx_b, rows_a, rows_b,
                     sem_a, sem_b):
    c = lax.axis_index("c")
    s = lax.axis_index("s")
    tbase = (c * NS + s) * TPT
    pltpu.sync_copy(idx_hbm.at[pl.ds(tbase, GCH)], idx_a)
    pltpu.sync_copy(idx_hbm.at[pl.ds(tbase + GCH, GCH)], idx_b)
    cp0 = pltpu.async_copy(src_hbm.at[idx_a], rows_a, sem_a)
    cp1 = pltpu.async_copy(src_hbm.at[idx_b], rows_b, sem_b)
    cp0.wait()
    pltpu.sync_copy(rows_a, out_hbm.at[pl.ds(tbase, GCH)])
    cp1.wait()
    pltpu.sync_copy(rows_b, out_hbm.at[pl.ds(tbase + GCH, GCH)])


def _sc_dispatch_body(pos_hbm, x_hbm, g_hbm, xs_hbm, gsort_hbm,
                      pos_v, g_v, tok_v, gso_v, rows_a, rows_b,
                      sem_a, sem_b):
    c = lax.axis_index("c")
    s = lax.axis_index("s")
    base = (c * NS + s) * RPT
    pltpu.sync_copy(pos_hbm, pos_v)
    pltpu.sync_copy(g_hbm, g_v)

    i16 = lax.iota(jnp.int32, 16)
    for j in range(RPT // GCH):
        for k in range(GCH // 16):
            sl = i16 + (base + j * GCH + k * 16)
            tok_v[j, pl.ds(k * 16, 16)] = jnp.where(sl >= T, sl - T, sl)

    def scan(i, carry):
        for u in range(4):
            o = i * 64 + u * 16
            pc = pos_v[pl.ds(o, 16)]
            li = pc - base
            m = (li >= 0) & (li < RPT)
            lic = jnp.minimum(jnp.maximum(li, 0), RPT - 1)
            plsc.store_scatter(tok_v, [lax.shift_right_logical(lic, 6),
                                       lic & (GCH - 1)], i16 + o, mask=m)
        return carry
    lax.fori_loop(0, T // 64, scan, 0)

    cp0 = pltpu.async_copy(x_hbm.at[tok_v.at[0]], rows_a, sem_a)
    cp1 = pltpu.async_copy(x_hbm.at[tok_v.at[1]], rows_b, sem_b)
    for j in range(RPT // GCH):
        for k in range(GCH // 16):
            tk = tok_v[j, pl.ds(k * 16, 16)]
            gso_v[pl.ds(j * GCH + k * 16, 16)] = plsc.load_gather(g_v, [tk])
    pltpu.sync_copy(gso_v, gsort_hbm.at[pl.ds(base, RPT)])
    cp0.wait()
    pltpu.sync_copy(rows_a, xs_hbm.at[pl.ds(base, GCH)])
    cp2 = pltpu.async_copy(x_hbm.at[tok_v.at[2]], rows_a, sem_a)
    cp1.wait()
    pltpu.sync_copy(rows_b, xs_hbm.at[pl.ds(base + GCH, GCH)])
    cp2.wait()
    pltpu.sync_copy(rows_a, xs_hbm.at[pl.ds(base + 2 * GCH, GCH)])


@functools.lru_cache(maxsize=1)
def _sc_kernels():
    mesh = plsc.VectorSubcoreMesh(core_axis_name="c", subcore_axis_name="s")

    dispatch = pl.kernel(
        _sc_dispatch_body,
        out_type=(
            jax.ShapeDtypeStruct((NP, H), jnp.float32),
            jax.ShapeDtypeStruct((NP,), jnp.float32),
        ),
        mesh=mesh,
        scratch_types=(
            pltpu.VMEM((T,), jnp.int32),
            pltpu.VMEM((T,), jnp.float32),
            pltpu.VMEM((RPT // GCH, GCH), jnp.int32),
            pltpu.VMEM((RPT,), jnp.float32),
            pltpu.VMEM((GCH, H), jnp.float32),
            pltpu.VMEM((GCH, H), jnp.float32),
            pltpu.SemaphoreType.DMA,
            pltpu.SemaphoreType.DMA,
        ),
        compiler_params=pltpu.CompilerParams(needs_layout_passes=False),
    )
    combine = pl.kernel(
        _sc_combine_body,
        out_type=jax.ShapeDtypeStruct((T, H), jnp.float32),
        mesh=mesh,
        scratch_types=(
            pltpu.VMEM((GCH,), jnp.int32),
            pltpu.VMEM((GCH,), jnp.int32),
            pltpu.VMEM((GCH, H), jnp.float32),
            pltpu.VMEM((GCH, H), jnp.float32),
            pltpu.SemaphoreType.DMA,
            pltpu.SemaphoreType.DMA,
        ),
        compiler_params=pltpu.CompilerParams(needs_layout_passes=False),
    )
    return dispatch, combine


@jax.jit
def kernel(x, Wg, W1, b1, W2, b2):
    x_flat = x.reshape(T, H)

    gs, pos2, be2, bv2, g2, loss = pl.pallas_call(
        _router_body,
        out_shape=(
            jax.ShapeDtypeStruct((T, E), jnp.float32),
            jax.ShapeDtypeStruct((T // 128, 128), jnp.int32),
            jax.ShapeDtypeStruct((NBP, 1), jnp.int32),
            jax.ShapeDtypeStruct((NBP, 1), jnp.int32),
            jax.ShapeDtypeStruct((T // 128, 128), jnp.float32),
            jax.ShapeDtypeStruct((1, 1), jnp.float32),
        ),
    )(x_flat, Wg)
    pos = pos2.reshape(T)

    _sc_dispatch, _sc_combine = _sc_kernels()
    x_sorted, g_sorted = _sc_dispatch(pos, x_flat, g2.reshape(T))

    y_sorted = pl.pallas_call(
        _ffn_body,
        grid_spec=pltpu.PrefetchScalarGridSpec(
            num_scalar_prefetch=2,
            grid=(NB,),
            in_specs=[
                pl.BlockSpec((TB, H), lambda b, be, bv: (b, 0)),
                pl.BlockSpec((1, F, H), lambda b, be, bv: (be[b], 0, 0)),
                pl.BlockSpec((1, 1, F), lambda b, be, bv: (be[b], 0, 0)),
                pl.BlockSpec((1, H, F), lambda b, be, bv: (be[b], 0, 0)),
                pl.BlockSpec((1, 1, H), lambda b, be, bv: (be[b], 0, 0)),
                pl.BlockSpec((TB, 1), lambda b, be, bv: (b, 0)),
            ],
            out_specs=pl.BlockSpec((TB, H), lambda b, be, bv: (b, 0)),
        ),
        out_shape=jax.ShapeDtypeStruct((NP, H), jnp.float32),
        compiler_params=pltpu.CompilerParams(
            dimension_semantics=("arbitrary",),
        ),
    )(be2.reshape(NBP), bv2.reshape(NBP), x_sorted, W1,
      b1.reshape(E, 1, F), W2, b2.reshape(E, 1, H),
      g_sorted.reshape(NP, 1))

    out_flat = _sc_combine(pos, y_sorted)

    return out_flat.reshape(B, S, H), gs.reshape(B, S, E), loss.reshape(())
